# Initial kernel scaffold; baseline (speedup 1.0000x reference)
import jax
import jax.numpy as jnp
from jax.experimental import pallas as pl


def kernel(delta_noisy, n, edge_index, W_in, b_in, Wt1, bt1, Wt2, bt2, W1s, b1s, g1s, be1s, W2s, b2s, gns, bns, W_out, b_out):
    raise NotImplementedError("write your pallas kernel here")


# speedup vs baseline: 3.5635x; 3.5635x over previous
import functools

import jax
import jax.numpy as jnp
from jax import lax
from jax.experimental import pallas as pl
from jax.experimental.pallas import tpu as pltpu
from jax.experimental.pallas import tpu_sc as plsc

E = 320000
V = 10000
D = 128
NC = 2
NS = 16
NW = NC * NS
EPW = E // NW
CH = 80
NCH = EPW // CH
VPS = 624
VPS_LAST = V - VPS * (NS - 1)

_MESH = dict(core_axis_name="c", subcore_axis_name="s")


def _sc_scatter(h, src_r, dst_r, zeros_vd):
  @functools.partial(
      pl.kernel,
      mesh=plsc.VectorSubcoreMesh(**_MESH),
      out_type=jax.ShapeDtypeStruct((NC, V, D), jnp.float32),
      scratch_types=[
          pltpu.VMEM((NCH, CH), jnp.int32),
          pltpu.VMEM((NCH, CH), jnp.int32),
          pltpu.VMEM((CH, D), jnp.float32),
          pltpu.VMEM_SHARED((V, D), jnp.float32),
      ],
  )
  def k(h_hbm, src_hbm, dst_hbm, z_hbm, out_hbm, idxs_v, idxd_v, row_v, vf_sh):
    c = lax.axis_index("c")
    s = lax.axis_index("s")
    wid = c * NS + s
    off = pl.multiple_of(s * VPS, 8)
    @pl.when(s < NS - 1)
    def _():
      pltpu.sync_copy(z_hbm.at[pl.ds(off, VPS)], vf_sh.at[pl.ds(off, VPS)])

    @pl.when(s == NS - 1)
    def _():
      pltpu.sync_copy(z_hbm.at[pl.ds((NS - 1) * VPS, VPS_LAST)],
                      vf_sh.at[pl.ds((NS - 1) * VPS, VPS_LAST)])
    pltpu.sync_copy(src_hbm.at[wid], idxs_v)
    pltpu.sync_copy(dst_hbm.at[wid], idxd_v)
    plsc.subcore_barrier()

    def chunk(ci, carry):
      base = pl.multiple_of(wid * EPW + ci * CH, CH)
      pltpu.sync_copy(h_hbm.at[pl.ds(base, CH)], row_v)
      pltpu.sync_copy(row_v, vf_sh.at[idxs_v.at[ci]], add=True)
      pltpu.sync_copy(row_v, vf_sh.at[idxd_v.at[ci]], add=True)
      return carry

    lax.fori_loop(0, NCH, chunk, 0)
    plsc.subcore_barrier()

    @pl.when(s < NS - 1)
    def _():
      pltpu.sync_copy(vf_sh.at[pl.ds(off, VPS)],
                      out_hbm.at[c, pl.ds(off, VPS)])

    @pl.when(s == NS - 1)
    def _():
      pltpu.sync_copy(vf_sh.at[pl.ds((NS - 1) * VPS, VPS_LAST)],
                      out_hbm.at[c, pl.ds((NS - 1) * VPS, VPS_LAST)])

  return k(h, src_r, dst_r, zeros_vd)


def _sc_gather(ta, tb, src_r, dst_r):
  @functools.partial(
      pl.kernel,
      mesh=plsc.VectorSubcoreMesh(**_MESH),
      out_type=jax.ShapeDtypeStruct((E, D), jnp.float32),
      scratch_types=[
          pltpu.VMEM((NCH, CH), jnp.int32),
          pltpu.VMEM((NCH, CH), jnp.int32),
          pltpu.VMEM((CH, D), jnp.float32),
          pltpu.VMEM((CH, D), jnp.float32),
          pltpu.SemaphoreType.DMA,
          pltpu.SemaphoreType.DMA,
      ],
  )
  def k(a_hbm, b_hbm, src_hbm, dst_hbm, z_hbm, idxs_v, idxd_v, bufa, bufb,
        sema, semb):
    c = lax.axis_index("c")
    s = lax.axis_index("s")
    wid = c * NS + s
    pltpu.sync_copy(src_hbm.at[wid], idxs_v)
    pltpu.sync_copy(dst_hbm.at[wid], idxd_v)

    def chunk(ci, carry):
      base = pl.multiple_of(wid * EPW + ci * CH, CH)
      cpa = pltpu.async_copy(a_hbm.at[idxs_v.at[ci]], bufa, sema)
      cpb = pltpu.async_copy(b_hbm.at[idxd_v.at[ci]], bufb, semb)
      cpa.wait()
      cpb.wait()

      def row(i, rc):
        for j in range(D // 16):
          sl = pl.ds(j * 16, 16)
          bufa[i, sl] = bufa[i, sl] + bufb[i, sl]
        return rc

      lax.fori_loop(0, CH, row, 0)
      pltpu.sync_copy(bufa, z_hbm.at[pl.ds(base, CH)])
      return carry

    lax.fori_loop(0, NCH, chunk, 0)

  return k(ta, tb, src_r, dst_r)


def _tc_temb(nf, Wt1, bt1, Wt2, bt2):
  def body(nf_ref, w1_ref, b1_ref, w2_ref, b2_ref, o_ref):
    nv = nf_ref[0, 0]
    j = lax.broadcasted_iota(jnp.int32, (1, D), 1).astype(jnp.float32)
    half = D // 2
    k = jnp.where(j < half, j, j - half)
    freq = jnp.exp(-jnp.log(10000.0) * k / half)
    arg = nv * freq
    emb = jnp.where(j < half, jnp.sin(arg), jnp.cos(arg))
    hdn = jnp.dot(emb, w1_ref[...], preferred_element_type=jnp.float32)
    hdn = hdn + b1_ref[...]
    hact = hdn * jax.nn.sigmoid(hdn)
    out = jnp.dot(hact, w2_ref[...], preferred_element_type=jnp.float32)
    o_ref[...] = out + b2_ref[...]

  return pl.pallas_call(
      body,
      out_shape=jax.ShapeDtypeStruct((1, D), jnp.float32),
  )(nf, Wt1, bt1.reshape(1, 4 * D), Wt2, bt2.reshape(1, D))


def _tc_proj(delta, W_in, b_in, temb):
  BE = 4000

  def body(d_ref, w_ref, b_ref, t_ref, o_ref):
    o_ref[...] = (
        jnp.dot(d_ref[...], w_ref[...], preferred_element_type=jnp.float32)
        + b_ref[...] + t_ref[...])

  return pl.pallas_call(
      body,
      grid=(E // BE,),
      in_specs=[
          pl.BlockSpec((BE, 3), lambda i: (i, 0)),
          pl.BlockSpec((3, D), lambda i: (0, 0)),
          pl.BlockSpec((1, D), lambda i: (0, 0)),
          pl.BlockSpec((1, D), lambda i: (0, 0)),
      ],
      out_specs=pl.BlockSpec((BE, D), lambda i: (i, 0)),
      out_shape=jax.ShapeDtypeStruct((E, D), jnp.float32),
  )(delta, W_in, b_in.reshape(1, D), temb)


def _tc_vertex(partials, W1a, W1b, b1):
  BV = 2000

  def body(p_ref, wa_ref, wb_ref, b1_ref, a_ref, b_ref):
    vf = p_ref[0] + p_ref[1]
    a_ref[...] = jnp.dot(vf, wa_ref[...], preferred_element_type=jnp.float32)
    b_ref[...] = (
        jnp.dot(vf, wb_ref[...], preferred_element_type=jnp.float32)
        + b1_ref[...])

  return pl.pallas_call(
      body,
      grid=(V // BV,),
      in_specs=[
          pl.BlockSpec((NC, BV, D), lambda i: (0, i, 0)),
          pl.BlockSpec((D, D), lambda i: (0, 0)),
          pl.BlockSpec((D, D), lambda i: (0, 0)),
          pl.BlockSpec((1, D), lambda i: (0, 0)),
      ],
      out_specs=[
          pl.BlockSpec((BV, D), lambda i: (i, 0)),
          pl.BlockSpec((BV, D), lambda i: (i, 0)),
      ],
      out_shape=[
          jax.ShapeDtypeStruct((V, D), jnp.float32),
          jax.ShapeDtypeStruct((V, D), jnp.float32),
      ],
  )(partials, W1a, W1b, b1.reshape(1, D))


def _ln(x, g, b):
  mu = jnp.mean(x, axis=-1, keepdims=True)
  var = jnp.mean((x - mu) ** 2, axis=-1, keepdims=True)
  return (x - mu) * lax.rsqrt(var + 1e-5) * g + b


def _tc_edge(z, h, g1, be1, W2, b2, gn, bn):
  BE = 4000

  def body(z_ref, h_ref, g1_ref, e1_ref, w2_ref, b2_ref, gn_ref, bn_ref,
           o_ref):
    zn = _ln(z_ref[...], g1_ref[...], e1_ref[...])
    sl = zn * jax.nn.sigmoid(zn)
    hn = jnp.dot(sl, w2_ref[...], preferred_element_type=jnp.float32)
    hn = hn + b2_ref[...]
    o_ref[...] = _ln(h_ref[...] + hn, gn_ref[...], bn_ref[...])

  vec = lambda: pl.BlockSpec((1, D), lambda i: (0, 0))
  return pl.pallas_call(
      body,
      grid=(E // BE,),
      in_specs=[
          pl.BlockSpec((BE, D), lambda i: (i, 0)),
          pl.BlockSpec((BE, D), lambda i: (i, 0)),
          vec(), vec(),
          pl.BlockSpec((D, D), lambda i: (0, 0)),
          vec(), vec(), vec(),
      ],
      out_specs=pl.BlockSpec((BE, D), lambda i: (i, 0)),
      out_shape=jax.ShapeDtypeStruct((E, D), jnp.float32),
  )(z, h, g1.reshape(1, D), be1.reshape(1, D), W2, b2.reshape(1, D),
    gn.reshape(1, D), bn.reshape(1, D))


def _tc_edge_final(z, h, g1, be1, W2, b2, gn, bn, W_out, b_out):
  BE = 4000

  def body(z_ref, h_ref, g1_ref, e1_ref, w2_ref, b2_ref, gn_ref, bn_ref,
           wo_ref, bo_ref, o_ref):
    zn = _ln(z_ref[...], g1_ref[...], e1_ref[...])
    sl = zn * jax.nn.sigmoid(zn)
    hn = jnp.dot(sl, w2_ref[...], preferred_element_type=jnp.float32)
    hn = hn + b2_ref[...]
    hf = _ln(h_ref[...] + hn, gn_ref[...], bn_ref[...])
    o_ref[...] = (
        jnp.dot(hf, wo_ref[...], preferred_element_type=jnp.float32)
        + bo_ref[...])

  vec = lambda: pl.BlockSpec((1, D), lambda i: (0, 0))
  return pl.pallas_call(
      body,
      grid=(E // BE,),
      in_specs=[
          pl.BlockSpec((BE, D), lambda i: (i, 0)),
          pl.BlockSpec((BE, D), lambda i: (i, 0)),
          vec(), vec(),
          pl.BlockSpec((D, D), lambda i: (0, 0)),
          vec(), vec(), vec(),
          pl.BlockSpec((D, 3), lambda i: (0, 0)),
          pl.BlockSpec((1, 3), lambda i: (0, 0)),
      ],
      out_specs=pl.BlockSpec((BE, 3), lambda i: (i, 0)),
      out_shape=jax.ShapeDtypeStruct((E, 3), jnp.float32),
  )(z, h, g1.reshape(1, D), be1.reshape(1, D), W2, b2.reshape(1, D),
    gn.reshape(1, D), bn.reshape(1, D), W_out, b_out.reshape(1, 3))


def kernel(delta_noisy, n, edge_index, W_in, b_in, Wt1, bt1, Wt2, bt2,
           W1s, b1s, g1s, be1s, W2s, b2s, gns, bns, W_out, b_out):
  L = W1s.shape[0]
  src_r = edge_index[0].reshape(NW, NCH, CH)
  dst_r = edge_index[1].reshape(NW, NCH, CH)
  zeros_vd = jnp.zeros((V, D), jnp.float32)

  temb = _tc_temb(n.astype(jnp.float32).reshape(1, 1), Wt1, bt1, Wt2, bt2)
  h = _tc_proj(delta_noisy[0], W_in, b_in, temb)

  for l in range(L):
    partials = _sc_scatter(h, src_r, dst_r, zeros_vd)
    ta, tb = _tc_vertex(partials, W1s[l, :D], W1s[l, D:], b1s[l])
    z = _sc_gather(ta, tb, src_r, dst_r)
    if l < L - 1:
      h = _tc_edge(z, h, g1s[l], be1s[l], W2s[l], b2s[l], gns[l], bns[l])
    else:
      out = _tc_edge_final(z, h, g1s[l], be1s[l], W2s[l], b2s[l],
                           gns[l], bns[l], W_out, b_out)
  return out[None]


# speedup vs baseline: 5.0529x; 1.4180x over previous
import functools

import jax
import jax.numpy as jnp
from jax import lax
from jax.experimental import pallas as pl
from jax.experimental.pallas import tpu as pltpu
from jax.experimental.pallas import tpu_sc as plsc

E = 320000
V = 10000
D = 128
NC = 2
NS = 16
NW = NC * NS
EPW = E // NW
CH = 80
NCH = EPW // CH
VPS = 624
VPS_LAST = V - VPS * (NS - 1)

_MESH = dict(core_axis_name="c", subcore_axis_name="s")
NBUF = 5


def _sc_scatter(h, src_r, dst_r, zeros_vd):
  NB = 4

  @functools.partial(
      pl.kernel,
      mesh=plsc.VectorSubcoreMesh(**_MESH),
      out_type=jax.ShapeDtypeStruct((NC, V, D), jnp.float32),
      scratch_types=(
          [pltpu.VMEM_SHARED((V, D), jnp.float32)]
          + [pltpu.VMEM((NB, CH), jnp.int32)] * 2
          + [pltpu.VMEM((CH, D), jnp.float32)] * NB
          + [pltpu.SemaphoreType.DMA] * NB
      ),
  )
  def k(h_hbm, src_hbm, dst_hbm, z_hbm, out_hbm, vf_sh, idxs_v, idxd_v, *rest):
    bufs = rest[:NB]
    lsem = rest[NB:2 * NB]
    c = lax.axis_index("c")
    s = lax.axis_index("s")
    wid = c * NS + s
    off = pl.multiple_of(s * VPS, 8)

    def start_load(ci, b):
      base = pl.multiple_of(wid * EPW + ci * CH, CH)
      pltpu.async_copy(h_hbm.at[pl.ds(base, CH)], bufs[b], lsem[b])
      pltpu.async_copy(src_hbm.at[wid, ci], idxs_v.at[b], lsem[b])
      pltpu.async_copy(dst_hbm.at[wid, ci], idxd_v.at[b], lsem[b])

    def wait_load(b):
      pltpu.make_async_copy(h_hbm.at[pl.ds(0, CH)], bufs[b], lsem[b]).wait()
      pltpu.make_async_copy(src_hbm.at[0, 0], idxs_v.at[b], lsem[b]).wait()
      pltpu.make_async_copy(dst_hbm.at[0, 0], idxd_v.at[b], lsem[b]).wait()

    def do_adds(b):
      pltpu.sync_copy(bufs[b], vf_sh.at[idxs_v.at[b]], add=True)
      pltpu.sync_copy(bufs[b], vf_sh.at[idxd_v.at[b]], add=True)

    @pl.when(s < NS - 1)
    def _():
      pltpu.sync_copy(z_hbm.at[pl.ds(off, VPS)], vf_sh.at[pl.ds(off, VPS)])

    @pl.when(s == NS - 1)
    def _():
      pltpu.sync_copy(z_hbm.at[pl.ds((NS - 1) * VPS, VPS_LAST)],
                      vf_sh.at[pl.ds((NS - 1) * VPS, VPS_LAST)])
    plsc.subcore_barrier()

    for b in range(NB - 2):
      start_load(b, b)

    def group(g, carry):
      for b in range(NB):
        ci = g * NB + b

        @pl.when(ci + NB - 2 < NCH)
        def _():
          start_load(ci + NB - 2, (b + NB - 2) % NB)
        wait_load(b)
        do_adds(b)
      return carry

    lax.fori_loop(0, NCH // NB, group, 0)
    for ci in range(NCH - NCH % NB, NCH):
      wait_load(ci % NB)
      do_adds(ci % NB)
    plsc.subcore_barrier()

    @pl.when(s < NS - 1)
    def _():
      pltpu.sync_copy(vf_sh.at[pl.ds(off, VPS)],
                      out_hbm.at[c, pl.ds(off, VPS)])

    @pl.when(s == NS - 1)
    def _():
      pltpu.sync_copy(vf_sh.at[pl.ds((NS - 1) * VPS, VPS_LAST)],
                      out_hbm.at[c, pl.ds((NS - 1) * VPS, VPS_LAST)])

  return k(h, src_r, dst_r, zeros_vd)


def _sc_gather(ta, tb, src_r, dst_r):
  @functools.partial(
      pl.kernel,
      mesh=plsc.VectorSubcoreMesh(**_MESH),
      out_type=jax.ShapeDtypeStruct((E, D), jnp.float32),
      scratch_types=(
          [pltpu.VMEM((NBUF, CH), jnp.int32)] * 2
          + [pltpu.VMEM((CH, D), jnp.float32)] * (2 * NBUF)
          + [pltpu.SemaphoreType.DMA] * (4 * NBUF)
      ),
  )
  def k(a_hbm, b_hbm, src_hbm, dst_hbm, z_hbm, idxs_v, idxd_v, *rest):
    bufa = rest[:NBUF]
    bufb = rest[NBUF:2 * NBUF]
    gsem = rest[2 * NBUF:3 * NBUF]
    hsem = rest[3 * NBUF:4 * NBUF]
    ssem = rest[4 * NBUF:5 * NBUF]
    isem = rest[5 * NBUF:6 * NBUF]
    c = lax.axis_index("c")
    s = lax.axis_index("s")
    wid = c * NS + s

    def start_idx(ci, b):
      pltpu.async_copy(src_hbm.at[wid, ci], idxs_v.at[b], isem[b])
      pltpu.async_copy(dst_hbm.at[wid, ci], idxd_v.at[b], isem[b])

    def wait_idx(b):
      pltpu.make_async_copy(src_hbm.at[0, 0], idxs_v.at[b], isem[b]).wait()
      pltpu.make_async_copy(dst_hbm.at[0, 0], idxd_v.at[b], isem[b]).wait()

    def start_gathers(b):
      pltpu.async_copy(a_hbm.at[idxs_v.at[b]], bufa[b], gsem[b])
      pltpu.async_copy(b_hbm.at[idxd_v.at[b]], bufb[b], hsem[b])

    def wait_gathers(b):
      pltpu.make_async_copy(a_hbm.at[idxs_v.at[b]], bufa[b], gsem[b]).wait()
      pltpu.make_async_copy(b_hbm.at[idxd_v.at[b]], bufb[b], hsem[b]).wait()

    def start_store(ci, b):
      base = pl.multiple_of(wid * EPW + ci * CH, CH)
      pltpu.async_copy(bufa[b], z_hbm.at[pl.ds(base, CH)], ssem[b])

    def wait_store(b):
      pltpu.make_async_copy(bufa[b], z_hbm.at[pl.ds(0, CH)], ssem[b]).wait()

    for b in range(3):
      start_idx(b, b)
    for b in range(2):
      wait_idx(b)
      start_gathers(b)

    def group(g, carry):
      for b in range(NBUF):
        ci = g * NBUF + b
        bn3 = (b + 3) % NBUF
        bn2 = (b + 2) % NBUF

        @pl.when(ci >= 2)
        def _():
          wait_store(bn3)

        @pl.when(ci + 3 < NCH)
        def _():
          start_idx(ci + 3, bn3)

        @pl.when(ci + 2 < NCH)
        def _():
          wait_idx(bn2)
          start_gathers(bn2)
        wait_gathers(b)

        def row(i, rc):
          for j in range(D // 16):
            sl = pl.ds(j * 16, 16)
            bufa[b][i, sl] = bufa[b][i, sl] + bufb[b][i, sl]
          return rc

        lax.fori_loop(0, CH, row, 0)
        start_store(ci, b)
      return carry

    lax.fori_loop(0, NCH // NBUF, group, 0)
    for t in range(2):
      wait_store((NCH - 2 + t) % NBUF)

  return k(ta, tb, src_r, dst_r)


def _tc_temb(nf, Wt1, bt1, Wt2, bt2):
  def body(nf_ref, w1_ref, b1_ref, w2_ref, b2_ref, o_ref):
    nv = nf_ref[0, 0]
    j = lax.broadcasted_iota(jnp.int32, (1, D), 1).astype(jnp.float32)
    half = D // 2
    k = jnp.where(j < half, j, j - half)
    freq = jnp.exp(-jnp.log(10000.0) * k / half)
    arg = nv * freq
    emb = jnp.where(j < half, jnp.sin(arg), jnp.cos(arg))
    hdn = jnp.dot(emb, w1_ref[...], preferred_element_type=jnp.float32)
    hdn = hdn + b1_ref[...]
    hact = hdn * jax.nn.sigmoid(hdn)
    out = jnp.dot(hact, w2_ref[...], preferred_element_type=jnp.float32)
    o_ref[...] = out + b2_ref[...]

  return pl.pallas_call(
      body,
      out_shape=jax.ShapeDtypeStruct((1, D), jnp.float32),
  )(nf, Wt1, bt1.reshape(1, 4 * D), Wt2, bt2.reshape(1, D))


def _tc_proj(delta, W_in, b_in, temb):
  BE = 4000

  def body(d_ref, w_ref, b_ref, t_ref, o_ref):
    o_ref[...] = (
        jnp.dot(d_ref[...], w_ref[...], preferred_element_type=jnp.float32)
        + b_ref[...] + t_ref[...])

  return pl.pallas_call(
      body,
      grid=(E // BE,),
      in_specs=[
          pl.BlockSpec((BE, 3), lambda i: (i, 0)),
          pl.BlockSpec((3, D), lambda i: (0, 0)),
          pl.BlockSpec((1, D), lambda i: (0, 0)),
          pl.BlockSpec((1, D), lambda i: (0, 0)),
      ],
      out_specs=pl.BlockSpec((BE, D), lambda i: (i, 0)),
      out_shape=jax.ShapeDtypeStruct((E, D), jnp.float32),
  )(delta, W_in, b_in.reshape(1, D), temb)


def _tc_vertex(partials, W1a, W1b, b1):
  BV = 2000

  def body(p_ref, wa_ref, wb_ref, b1_ref, a_ref, b_ref):
    vf = p_ref[0] + p_ref[1]
    a_ref[...] = jnp.dot(vf, wa_ref[...], preferred_element_type=jnp.float32)
    b_ref[...] = (
        jnp.dot(vf, wb_ref[...], preferred_element_type=jnp.float32)
        + b1_ref[...])

  return pl.pallas_call(
      body,
      grid=(V // BV,),
      in_specs=[
          pl.BlockSpec((NC, BV, D), lambda i: (0, i, 0)),
          pl.BlockSpec((D, D), lambda i: (0, 0)),
          pl.BlockSpec((D, D), lambda i: (0, 0)),
          pl.BlockSpec((1, D), lambda i: (0, 0)),
      ],
      out_specs=[
          pl.BlockSpec((BV, D), lambda i: (i, 0)),
          pl.BlockSpec((BV, D), lambda i: (i, 0)),
      ],
      out_shape=[
          jax.ShapeDtypeStruct((V, D), jnp.float32),
          jax.ShapeDtypeStruct((V, D), jnp.float32),
      ],
  )(partials, W1a, W1b, b1.reshape(1, D))


def _ln(x, g, b):
  mu = jnp.mean(x, axis=-1, keepdims=True)
  var = jnp.mean((x - mu) ** 2, axis=-1, keepdims=True)
  return (x - mu) * lax.rsqrt(var + 1e-5) * g + b


def _tc_edge(z, h, g1, be1, W2, b2, gn, bn):
  BE = 4000

  def body(z_ref, h_ref, g1_ref, e1_ref, w2_ref, b2_ref, gn_ref, bn_ref,
           o_ref):
    zn = _ln(z_ref[...], g1_ref[...], e1_ref[...])
    sl = zn * jax.nn.sigmoid(zn)
    hn = jnp.dot(sl, w2_ref[...], preferred_element_type=jnp.float32)
    hn = hn + b2_ref[...]
    o_ref[...] = _ln(h_ref[...] + hn, gn_ref[...], bn_ref[...])

  vec = lambda: pl.BlockSpec((1, D), lambda i: (0, 0))
  return pl.pallas_call(
      body,
      grid=(E // BE,),
      in_specs=[
          pl.BlockSpec((BE, D), lambda i: (i, 0)),
          pl.BlockSpec((BE, D), lambda i: (i, 0)),
          vec(), vec(),
          pl.BlockSpec((D, D), lambda i: (0, 0)),
          vec(), vec(), vec(),
      ],
      out_specs=pl.BlockSpec((BE, D), lambda i: (i, 0)),
      out_shape=jax.ShapeDtypeStruct((E, D), jnp.float32),
  )(z, h, g1.reshape(1, D), be1.reshape(1, D), W2, b2.reshape(1, D),
    gn.reshape(1, D), bn.reshape(1, D))


def _tc_edge_final(z, h, g1, be1, W2, b2, gn, bn, W_out, b_out):
  BE = 4000

  def body(z_ref, h_ref, g1_ref, e1_ref, w2_ref, b2_ref, gn_ref, bn_ref,
           wo_ref, bo_ref, o_ref):
    zn = _ln(z_ref[...], g1_ref[...], e1_ref[...])
    sl = zn * jax.nn.sigmoid(zn)
    hn = jnp.dot(sl, w2_ref[...], preferred_element_type=jnp.float32)
    hn = hn + b2_ref[...]
    hf = _ln(h_ref[...] + hn, gn_ref[...], bn_ref[...])
    o_ref[...] = (
        jnp.dot(hf, wo_ref[...], preferred_element_type=jnp.float32)
        + bo_ref[...])

  vec = lambda: pl.BlockSpec((1, D), lambda i: (0, 0))
  return pl.pallas_call(
      body,
      grid=(E // BE,),
      in_specs=[
          pl.BlockSpec((BE, D), lambda i: (i, 0)),
          pl.BlockSpec((BE, D), lambda i: (i, 0)),
          vec(), vec(),
          pl.BlockSpec((D, D), lambda i: (0, 0)),
          vec(), vec(), vec(),
          pl.BlockSpec((D, 3), lambda i: (0, 0)),
          pl.BlockSpec((1, 3), lambda i: (0, 0)),
      ],
      out_specs=pl.BlockSpec((BE, 3), lambda i: (i, 0)),
      out_shape=jax.ShapeDtypeStruct((E, 3), jnp.float32),
  )(z, h, g1.reshape(1, D), be1.reshape(1, D), W2, b2.reshape(1, D),
    gn.reshape(1, D), bn.reshape(1, D), W_out, b_out.reshape(1, 3))


def kernel(delta_noisy, n, edge_index, W_in, b_in, Wt1, bt1, Wt2, bt2,
           W1s, b1s, g1s, be1s, W2s, b2s, gns, bns, W_out, b_out):
  L = W1s.shape[0]
  src_r = edge_index[0].reshape(NW, NCH, CH)
  dst_r = edge_index[1].reshape(NW, NCH, CH)
  zeros_vd = jnp.zeros((V, D), jnp.float32)

  temb = _tc_temb(n.astype(jnp.float32).reshape(1, 1), Wt1, bt1, Wt2, bt2)
  h = _tc_proj(delta_noisy[0], W_in, b_in, temb)

  for l in range(L):
    partials = _sc_scatter(h, src_r, dst_r, zeros_vd)
    ta, tb = _tc_vertex(partials, W1s[l, :D], W1s[l, D:], b1s[l])
    z = _sc_gather(ta, tb, src_r, dst_r)
    if l < L - 1:
      h = _tc_edge(z, h, g1s[l], be1s[l], W2s[l], b2s[l], gns[l], bns[l])
    else:
      out = _tc_edge_final(z, h, g1s[l], be1s[l], W2s[l], b2s[l],
                           gns[l], bns[l], W_out, b_out)
  return out[None]
